# baseline (speedup 1.0000x reference)
import functools

import jax
import jax.numpy as jnp
from jax import lax
from jax.experimental import pallas as pl
from jax.experimental.pallas import tpu as pltpu
from jax.experimental.pallas import tpu_sc as plsc

_NC = 2
_NS = 16
_NW = _NC * _NS
_L = 16
_CH = 160


@functools.lru_cache(maxsize=None)
def _build_gather(B, D):
    b_per_w = B // _NW
    n_ch = b_per_w // _CH
    assert n_ch % 2 == 0
    mesh = plsc.VectorSubcoreMesh(core_axis_name="c", subcore_axis_name="s")

    @functools.partial(
        pl.kernel,
        mesh=mesh,
        out_type=jax.ShapeDtypeStruct((B, D), jnp.float32),
        compiler_params=pltpu.CompilerParams(needs_layout_passes=False),
        scratch_types=[
            pltpu.VMEM((b_per_w,), jnp.int32),
            pltpu.VMEM((b_per_w,), jnp.int32),
            pltpu.VMEM((2, _CH, 2 * D), jnp.float32),
            pltpu.VMEM((2, _CH, D), jnp.float32),
            pltpu.SemaphoreType.DMA,
            pltpu.SemaphoreType.DMA,
            pltpu.SemaphoreType.DMA,
            pltpu.SemaphoreType.DMA,
        ],
    )
    def gather_kernel(ids_hbm, table_hbm, out_hbm,
                      idx_v, pidx_v, slab_v, sel_v, g0, g1, s0, s1):
        wid = lax.axis_index("s") * _NC + lax.axis_index("c")
        base = wid * b_per_w
        pltpu.sync_copy(ids_hbm.at[pl.ds(base, b_per_w)], idx_v)

        def pre(i, _):
            v = idx_v[pl.ds(i * _L, _L)]
            pidx_v[pl.ds(i * _L, _L)] = v >> 1
            return ()

        lax.fori_loop(0, b_per_w // _L, pre, ())

        gsem = [g0, g1]
        ssem = [s0, s1]
        iota = lax.iota(jnp.int32, _L)
        jvs = [jb * _L + iota for jb in range(_CH // _L)]

        def start_gather(c, slot):
            pltpu.async_copy(
                table_hbm.at[pidx_v.at[pl.ds(c * _CH, _CH)]],
                slab_v.at[slot],
                gsem[slot],
            )

        def wait_gather(slot):
            pltpu.make_async_copy(
                table_hbm.at[pl.ds(0, _CH)], slab_v.at[slot], gsem[slot]
            ).wait()

        def start_store(c, slot):
            pltpu.async_copy(
                sel_v.at[slot],
                out_hbm.at[pl.ds(base + c * _CH, _CH)],
                ssem[slot],
            )

        def wait_store(slot):
            pltpu.make_async_copy(
                sel_v.at[slot], out_hbm.at[pl.ds(base, _CH)], ssem[slot]
            ).wait()

        def select(c, slot):
            for jb in range(_CH // _L):
                jv = jvs[jb]
                hv = idx_v[pl.ds(c * _CH + jb * _L, _L)]
                cv0 = (hv & 1) << 6
                for k in range(D):
                    kv = jnp.full((_L,), k, jnp.int32)
                    x = plsc.load_gather(slab_v.at[slot], [jv, cv0 + kv])
                    plsc.store_scatter(sel_v.at[slot], [jv, kv], x)

        start_gather(0, 0)
        start_gather(1, 1)

        def body(g, _):
            for b in range(2):
                c = 2 * g + b
                wait_gather(b)

                @pl.when(c >= 2)
                def _():
                    wait_store(b)

                select(c, b)

                @pl.when(c + 2 < n_ch)
                def _():
                    start_gather(c + 2, b)

                start_store(c, b)
            return ()

        lax.fori_loop(0, n_ch // 2, body, ())
        wait_store(0)
        wait_store(1)

    return gather_kernel


def kernel(input_ids, weight):
    batch, hist = input_ids.shape
    B = batch * hist
    V, D = weight.shape
    flat_ids = input_ids.reshape(B).astype(jnp.int32)
    packed = weight.reshape(V // 2, 2 * D)
    out = _build_gather(B, D)(flat_ids, packed)
    return out.reshape(batch, hist, D)


# speedup vs baseline: 1.4520x; 1.4520x over previous
import functools

import jax
import jax.numpy as jnp
from jax import lax
from jax.experimental import pallas as pl
from jax.experimental.pallas import tpu as pltpu
from jax.experimental.pallas import tpu_sc as plsc

_NC = 2
_NS = 16
_NW = _NC * _NS
_L = 16
_CH = 160


@functools.lru_cache(maxsize=None)
def _build_gather(B, D):
    b_per_w = B // _NW
    n_ch = b_per_w // _CH
    assert n_ch % 2 == 0
    mesh = plsc.VectorSubcoreMesh(core_axis_name="c", subcore_axis_name="s")

    @functools.partial(
        pl.kernel,
        mesh=mesh,
        out_type=jax.ShapeDtypeStruct((B * D,), jnp.float32),
        compiler_params=pltpu.CompilerParams(needs_layout_passes=False),
        scratch_types=[
            pltpu.VMEM((b_per_w,), jnp.int32),
            pltpu.VMEM((b_per_w,), jnp.int32),
            pltpu.VMEM((_CH, 2 * D), jnp.float32),
            pltpu.VMEM((_CH, 2 * D), jnp.float32),
            pltpu.VMEM((_CH * D,), jnp.float32),
            pltpu.VMEM((_CH * D,), jnp.float32),
            pltpu.SemaphoreType.DMA,
            pltpu.SemaphoreType.DMA,
            pltpu.SemaphoreType.DMA,
            pltpu.SemaphoreType.DMA,
        ],
    )
    def gather_kernel(ids_hbm, table_hbm, out_hbm,
                      idx_v, pidx_v, slab_a, slab_b, sel_a, sel_b,
                      g0, g1, s0, s1):
        wid = lax.axis_index("s") * _NC + lax.axis_index("c")
        base = wid * b_per_w
        pltpu.sync_copy(ids_hbm.at[pl.ds(base, b_per_w)], idx_v)

        def pre(i, _):
            v = idx_v[pl.ds(i * _L, _L)]
            pidx_v[pl.ds(i * _L, _L)] = v >> 1
            return ()

        lax.fori_loop(0, b_per_w // _L, pre, ())

        gsem = [g0, g1]
        ssem = [s0, s1]
        slabs = [slab_a, slab_b]
        sels = [sel_a, sel_b]
        iota = lax.iota(jnp.int32, _L)
        jvs = [jb * _L + iota for jb in range(_CH // _L)]

        def start_gather(c, slot):
            pltpu.async_copy(
                table_hbm.at[pidx_v.at[pl.ds(c * _CH, _CH)]],
                slabs[slot],
                gsem[slot],
            )

        def wait_gather(slot):
            pltpu.make_async_copy(
                table_hbm.at[pl.ds(0, _CH)], slabs[slot], gsem[slot]
            ).wait()

        def start_store(c, slot):
            pltpu.async_copy(
                sels[slot],
                out_hbm.at[pl.ds((base + c * _CH) * D, _CH * D)],
                ssem[slot],
            )

        def wait_store(slot):
            pltpu.make_async_copy(
                sels[slot], out_hbm.at[pl.ds(0, _CH * D)], ssem[slot]
            ).wait()

        def select(c, slot):
            for jb in range(_CH // _L):
                jv = jvs[jb]
                hv = idx_v[pl.ds(c * _CH + jb * _L, _L)]
                hb = (hv & 1) << 6
                ov0 = jv * D

                def kbody(q, _):
                    for u in range(4):
                        kv = (q * 4 + u + iota) & (D - 1)
                        x = plsc.load_gather(slabs[slot], [jv, hb + kv])
                        plsc.store_scatter(sels[slot], [ov0 + kv], x)
                    return ()

                lax.fori_loop(0, D // 4, kbody, ())

        start_gather(0, 0)
        start_gather(1, 1)

        def body(g, _):
            for b in range(2):
                c = 2 * g + b
                wait_gather(b)

                @pl.when(c >= 2)
                def _():
                    wait_store(b)

                select(c, b)

                @pl.when(c + 2 < n_ch)
                def _():
                    start_gather(c + 2, b)

                start_store(c, b)
            return ()

        lax.fori_loop(0, n_ch // 2, body, ())
        wait_store(0)
        wait_store(1)

    return gather_kernel


def kernel(input_ids, weight):
    batch, hist = input_ids.shape
    B = batch * hist
    V, D = weight.shape
    flat_ids = input_ids.reshape(B).astype(jnp.int32)
    packed = lax.optimization_barrier(weight.reshape(V // 2, 2 * D))
    out = _build_gather(B, D)(flat_ids, packed)
    return out.reshape(batch, hist, D)


# speedup vs baseline: 1.5965x; 1.0995x over previous
import functools

import jax
import jax.numpy as jnp
from jax import lax
from jax.experimental import pallas as pl
from jax.experimental.pallas import tpu as pltpu
from jax.experimental.pallas import tpu_sc as plsc

_NC = 2
_NS = 16
_NW = _NC * _NS
_CH = 800


@functools.lru_cache(maxsize=None)
def _build_gather(B, D):
    b_per_w = B // _NW
    n_ch = b_per_w // _CH
    mesh = plsc.VectorSubcoreMesh(core_axis_name="c", subcore_axis_name="s")

    @functools.partial(
        pl.kernel,
        mesh=mesh,
        out_type=jax.ShapeDtypeStruct((B, D), jnp.float32),
        compiler_params=pltpu.CompilerParams(use_tc_tiling_on_sc=False),
        scratch_types=[
            pltpu.VMEM((b_per_w,), jnp.int32),
            pltpu.VMEM((_CH, D), jnp.float32),
            pltpu.VMEM((_CH, D), jnp.float32),
            pltpu.SemaphoreType.DMA,
            pltpu.SemaphoreType.DMA,
            pltpu.SemaphoreType.DMA,
            pltpu.SemaphoreType.DMA,
        ],
    )
    def gather_kernel(ids_hbm, table_hbm, out_hbm, idx_v, slab_a, slab_b,
                      g0, g1, s0, s1):
        wid = lax.axis_index("s") * _NC + lax.axis_index("c")
        base = wid * b_per_w
        pltpu.sync_copy(ids_hbm.at[pl.ds(base, b_per_w)], idx_v)

        gsem = [g0, g1]
        ssem = [s0, s1]
        slabs = [slab_a, slab_b]
        gathers = [None, None]
        stores = [None, None]

        def start_gather(c):
            slot = c & 1
            gathers[slot] = pltpu.async_copy(
                table_hbm.at[idx_v.at[pl.ds(c * _CH, _CH)]],
                slabs[slot],
                gsem[slot],
            )

        start_gather(0)
        for c in range(n_ch):
            slot = c & 1
            if c + 1 < n_ch:
                nslot = (c + 1) & 1
                if stores[nslot] is not None:
                    stores[nslot].wait()
                    stores[nslot] = None
                start_gather(c + 1)
            gathers[slot].wait()
            stores[slot] = pltpu.async_copy(
                slabs[slot],
                out_hbm.at[pl.ds(base + c * _CH, _CH)],
                ssem[slot],
            )
        for slot in range(2):
            if stores[slot] is not None:
                stores[slot].wait()

    return gather_kernel


def kernel(input_ids, weight):
    batch, hist = input_ids.shape
    B = batch * hist
    D = weight.shape[1]
    flat_ids = input_ids.reshape(B).astype(jnp.int32)
    out = _build_gather(B, D)(flat_ids, weight)
    return out.reshape(batch, hist, D)


# speedup vs baseline: 1.6007x; 1.0026x over previous
import functools

import jax
import jax.numpy as jnp
from jax import lax
from jax.experimental import pallas as pl
from jax.experimental.pallas import tpu as pltpu
from jax.experimental.pallas import tpu_sc as plsc

_NC = 2
_NS = 16
_NW = _NC * _NS
_CH = 800


@functools.lru_cache(maxsize=None)
def _build_gather(B, D):
    b_per_w = B // _NW
    n_ch = b_per_w // _CH
    mesh = plsc.VectorSubcoreMesh(core_axis_name="c", subcore_axis_name="s")

    @functools.partial(
        pl.kernel,
        mesh=mesh,
        out_type=jax.ShapeDtypeStruct((B, D), jnp.float32),
        compiler_params=pltpu.CompilerParams(use_tc_tiling_on_sc=False),
        scratch_types=[
            pltpu.VMEM((b_per_w,), jnp.int32),
            pltpu.VMEM((_CH, D), jnp.float32),
            pltpu.VMEM((_CH, D), jnp.float32),
            pltpu.SemaphoreType.DMA,
            pltpu.SemaphoreType.DMA,
            pltpu.SemaphoreType.DMA,
            pltpu.SemaphoreType.DMA,
        ],
    )
    def gather_kernel(ids_hbm, table_hbm, out_hbm, idx_v, slab_a, slab_b,
                      g0, g1, s0, s1):
        wid = lax.axis_index("s") * _NC + lax.axis_index("c")
        base = wid * b_per_w
        pltpu.sync_copy(ids_hbm.at[pl.ds(base, b_per_w)], idx_v)

        gsem = [g0, g1]
        ssem = [s0, s1]
        slabs = [slab_a, slab_b]
        gathers = [None, None]
        stores = [None, None]

        def start_gather(c):
            slot = c & 1
            gathers[slot] = pltpu.async_copy(
                table_hbm.at[idx_v.at[pl.ds(c * _CH, _CH)]],
                slabs[slot],
                gsem[slot],
            )

        start_gather(0)
        for c in range(n_ch):
            slot = c & 1
            if c + 1 < n_ch:
                nslot = (c + 1) & 1
                if stores[nslot] is not None:
                    stores[nslot].wait()
                    stores[nslot] = None
                start_gather(c + 1)
            gathers[slot].wait()
            stores[slot] = pltpu.async_copy(
                slabs[slot],
                out_hbm.at[pl.ds(base + c * _CH, _CH)],
                ssem[slot],
            )
        for slot in range(2):
            if stores[slot] is not None:
                stores[slot].wait()

    return gather_kernel


def kernel(input_ids, weight):
    batch, hist = input_ids.shape
    B = batch * hist
    D = weight.shape[1]
    flat_ids = input_ids.reshape(B).astype(jnp.int32)
    s = jnp.sum(flat_ids[:8]) & jnp.int32(0)
    one = (s + 1).astype(jnp.float32)
    out = _build_gather(B, D)(flat_ids, weight * one)
    return out.reshape(batch, hist, D)
